# baseline (speedup 1.0000x reference)
import functools

import jax
import jax.numpy as jnp
from jax import lax
from jax.experimental import pallas as pl
from jax.experimental.pallas import tpu as pltpu
from jax.experimental.pallas import tpu_sc as plsc

_NUM = 16
_G = 8
_NC = 2
_NS = 16


def _tc_body(idx_ref, in_ref, out_ref):
    out_ref[...] = in_ref[...]


def kernel(x):
    b, c, t, h, w = x.shape
    bc = b * c
    rows_sc = _G * _NUM
    nw = _NC * _NS
    per = rows_sc // nw

    xr = x.reshape(bc * t, h, w)
    mesh = plsc.VectorSubcoreMesh(
        core_axis_name="c", subcore_axis_name="s",
        num_cores=_NC, num_subcores=_NS,
    )

    @functools.partial(
        pl.kernel,
        out_type=jax.ShapeDtypeStruct((rows_sc, h, w), x.dtype),
        mesh=mesh,
        scratch_types=[
            pltpu.VMEM((2, h, w), jnp.float32),
            pltpu.SemaphoreType.DMA,
            pltpu.SemaphoreType.DMA,
            pltpu.SemaphoreType.DMA,
        ],
    )
    def sc_gather(x_hbm, out_hbm, buf, sem_in, sem_out0, sem_out1):
        wid = lax.axis_index("s") * _NC + lax.axis_index("c")
        base = wid * per
        sems_out = (sem_out0, sem_out1)

        def fetch(k):
            r = base + k
            g = r // _NUM
            j = r - g * _NUM
            src = g * t + (j * (t - 1)) // (_NUM - 1)
            return pltpu.make_async_copy(x_hbm.at[src], buf.at[k % 2], sem_in)

        def store(k):
            return pltpu.make_async_copy(
                buf.at[k % 2], out_hbm.at[base + k], sems_out[k % 2])

        stores = [None] * per
        fetch(0).start()
        for k in range(per):
            fetch(k).wait()
            stores[k] = store(k)
            stores[k].start()
            if k + 1 < per:
                if k >= 1:
                    stores[k - 1].wait()
                fetch(k + 1).start()
        stores[per - 1].wait()

    sc_part = sc_gather(xr)

    idx = jnp.clip(jnp.linspace(0.0, t - 1, _NUM), 0, t - 1).astype(jnp.int32)
    xr4 = x.reshape(bc, t, h, w)
    ntb = (bc - _G) // _G
    out_tc = pl.pallas_call(
        _tc_body,
        grid_spec=pltpu.PrefetchScalarGridSpec(
            num_scalar_prefetch=1,
            grid=(ntb, _NUM),
            in_specs=[
                pl.BlockSpec((_G, 1, h, w),
                             lambda i, j, idx_ref: (i + 1, idx_ref[j], 0, 0)),
            ],
            out_specs=pl.BlockSpec((_G, 1, h, w),
                                   lambda i, j, idx_ref: (i + 1, j, 0, 0)),
        ),
        out_shape=jax.ShapeDtypeStruct((bc, _NUM, h, w), x.dtype),
    )(idx, xr4)

    out = lax.dynamic_update_slice(
        out_tc, sc_part.reshape(_G, _NUM, h, w), (0, 0, 0, 0))
    return out.reshape(b, c, _NUM, h, w)
